# Initial kernel scaffold; baseline (speedup 1.0000x reference)
import jax
import jax.numpy as jnp
from jax.experimental import pallas as pl


def kernel(x, w_cv1, bn_cv1_g, bn_cv1_b, bn_cv1_m, bn_cv1_v, w_m1, bn_m1_g, bn_m1_b, bn_m1_m, bn_m1_v, w_m2, bn_m2_g, bn_m2_b, bn_m2_m, bn_m2_v, w_cv3, w_cv2, bn_csp_g, bn_csp_b, bn_csp_m, bn_csp_v, w_cv4, bn_cv4_g, bn_cv4_b, bn_cv4_m, bn_cv4_v):
    raise NotImplementedError("write your pallas kernel here")


# speedup vs baseline: 1.1460x; 1.1460x over previous
import functools

import jax
import jax.numpy as jnp
from jax.experimental import pallas as pl
from jax.experimental.pallas import tpu as pltpu


_TAPS = tuple((dy, dx) for dy in (-1, 0, 1) for dx in (-1, 0, 1))


def _hswish(v):
    return v * jnp.clip(v + 3.0, 0.0, 6.0) * (1.0 / 6.0)


def _lrelu(v):
    return jnp.maximum(v, 0.1 * v)


def _csp_body(x_ref, mask_ref, wa_ref, ba_ref, wm1_ref, bm1_ref,
              wm2_ref, bm2_ref, wcv3_ref, bcspa_ref, wcv4_ref, bcv4_ref,
              out_ref, *, W, n, c_):
    f32 = jnp.float32
    bf16 = jnp.bfloat16
    x = x_ref[0]
    P = x.shape[1]
    dot = lambda a, b: jnp.dot(a, b, preferred_element_type=f32)

    tap_masks = [mask_ref[k] for k in range(8)]

    u = dot(wa_ref[...], x) + ba_ref[...]
    h32 = _hswish(u[:c_])
    z2 = _lrelu(u[c_:])

    for i in range(n):
        t = _hswish(dot(wm1_ref[i], h32.astype(bf16)) + bm1_ref[i])
        t = t.astype(bf16)
        taps = []
        mi = 0
        for dy, dx in _TAPS:
            if dy == 0 and dx == 0:
                taps.append(t)
            else:
                s = dy * W + dx
                sh = pltpu.roll(t, (-s) % P, axis=1)
                taps.append(sh * tap_masks[mi])
                mi += 1
        taps = jnp.concatenate(taps, axis=0)
        h32 = h32 + _hswish(dot(wm2_ref[i], taps) + bm2_ref[i])

    z1 = _lrelu(dot(wcv3_ref[...], h32.astype(bf16)) + bcspa_ref[...])
    zc = jnp.concatenate([z1, z2], axis=0).astype(bf16)
    o = _hswish(dot(wcv4_ref[...], zc) + bcv4_ref[...])
    out_ref[0] = o.astype(out_ref.dtype)


def _bn_fold(g, b, m, v, eps=1e-5):
    s = g / jnp.sqrt(v + eps)
    return s, b - m * s


def kernel(x, w_cv1, bn_cv1_g, bn_cv1_b, bn_cv1_m, bn_cv1_v,
           w_m1, bn_m1_g, bn_m1_b, bn_m1_m, bn_m1_v,
           w_m2, bn_m2_g, bn_m2_b, bn_m2_m, bn_m2_v,
           w_cv3, w_cv2, bn_csp_g, bn_csp_b, bn_csp_m, bn_csp_v,
           w_cv4, bn_cv4_g, bn_cv4_b, bn_cv4_m, bn_cv4_v):
    B, c1, H, W = x.shape
    P = H * W
    c_ = w_cv1.shape[0]
    c2 = w_cv4.shape[0]
    n = w_m1.shape[0]
    bf16 = jnp.bfloat16

    s1, b1 = _bn_fold(bn_cv1_g, bn_cv1_b, bn_cv1_m, bn_cv1_v)
    wk_cv1 = w_cv1[:, :, 0, 0] * s1[:, None]
    s_csp, b_csp = _bn_fold(bn_csp_g, bn_csp_b, bn_csp_m, bn_csp_v)
    wk_cv2 = w_cv2[:, :, 0, 0] * s_csp[c_:, None]
    wa = jnp.concatenate([wk_cv1, wk_cv2], axis=0).astype(bf16)
    ba = jnp.concatenate([b1[:, None], b_csp[c_:, None]], axis=0)

    sm1, bm1 = _bn_fold(bn_m1_g, bn_m1_b, bn_m1_m, bn_m1_v)
    wk_m1 = (w_m1[:, :, :, 0, 0] * sm1[:, :, None]).astype(bf16)
    bk_m1 = bm1[:, :, None]

    sm2, bm2 = _bn_fold(bn_m2_g, bn_m2_b, bn_m2_m, bn_m2_v)
    w2f = jnp.transpose(w_m2, (0, 1, 3, 4, 2)).reshape(n, c_, 9 * c_)
    wk_m2 = (w2f * sm2[:, :, None]).astype(bf16)
    bk_m2 = bm2[:, :, None]

    wk_cv3 = (w_cv3[:, :, 0, 0] * s_csp[:c_, None]).astype(bf16)
    bk_cspa = b_csp[:c_, None]

    s4, b4 = _bn_fold(bn_cv4_g, bn_cv4_b, bn_cv4_m, bn_cv4_v)
    wk_cv4 = (w_cv4[:, :, 0, 0] * s4[:, None]).astype(bf16)
    bk_cv4 = b4[:, None]

    pidx = jnp.arange(P, dtype=jnp.int32)
    prow, pcol = pidx // W, pidx % W
    masks = []
    for dy, dx in _TAPS:
        if dy == 0 and dx == 0:
            continue
        ok = ((prow + dy >= 0) & (prow + dy <= H - 1) &
              (pcol + dx >= 0) & (pcol + dx <= W - 1))
        masks.append(ok.astype(bf16).reshape(1, P))
    tap_masks = jnp.stack(masks)

    x_flat = x.reshape(B, c1, P).astype(bf16)

    body = functools.partial(_csp_body, W=W, n=n, c_=c_)
    rep2 = lambda b: (0, 0)
    rep3 = lambda b: (0, 0, 0)

    out_flat = pl.pallas_call(
        body,
        out_shape=jax.ShapeDtypeStruct((B, c2, P), jnp.float32),
        grid_spec=pltpu.PrefetchScalarGridSpec(
            num_scalar_prefetch=0,
            grid=(B,),
            in_specs=[
                pl.BlockSpec((1, c1, P), lambda b: (b, 0, 0)),
                pl.BlockSpec((8, 1, P), rep3),
                pl.BlockSpec((2 * c_, c1), rep2),
                pl.BlockSpec((2 * c_, 1), rep2),
                pl.BlockSpec((n, c_, c_), rep3),
                pl.BlockSpec((n, c_, 1), rep3),
                pl.BlockSpec((n, c_, 9 * c_), rep3),
                pl.BlockSpec((n, c_, 1), rep3),
                pl.BlockSpec((c_, c_), rep2),
                pl.BlockSpec((c_, 1), rep2),
                pl.BlockSpec((c2, 2 * c_), rep2),
                pl.BlockSpec((c2, 1), rep2),
            ],
            out_specs=pl.BlockSpec((1, c2, P), lambda b: (b, 0, 0)),
        ),
        compiler_params=pltpu.CompilerParams(
            dimension_semantics=("parallel",),
            vmem_limit_bytes=64 * 1024 * 1024),
    )(x_flat, tap_masks, wa, ba, wk_m1, bk_m1, wk_m2, bk_m2,
      wk_cv3, bk_cspa, wk_cv4, bk_cv4)

    return out_flat.reshape(B, c2, H, W)


# speedup vs baseline: 1.3628x; 1.1892x over previous
import functools

import jax
import jax.numpy as jnp
from jax.experimental import pallas as pl
from jax.experimental.pallas import tpu as pltpu


_TAPS = tuple((dy, dx) for dy in (-1, 0, 1) for dx in (-1, 0, 1))


def _hswish(v):
    return v * jnp.clip(v + 3.0, 0.0, 6.0) * (1.0 / 6.0)


def _lrelu(v):
    return jnp.maximum(v, 0.1 * v)


_IMGS = 2


def _csp_body(x_ref, mask_ref, wa_ref, ba_ref, wm1_ref, bm1_ref,
              wm2_ref, bm2_ref, wcv3_ref, bcspa_ref, wcv4_ref, bcv4_ref,
              out_ref, *, W, n, c_):
    f32 = jnp.float32
    bf16 = jnp.bfloat16
    dot = lambda a, b: jnp.dot(a, b, preferred_element_type=f32)
    tap_masks = [mask_ref[k] for k in range(8)]
    P = tap_masks[0].shape[-1]

    def s_in(j):
        u = dot(wa_ref[...], x_ref[j]) + ba_ref[...]
        return _hswish(u[:c_]), _lrelu(u[c_:])

    def s_m1(i, h32):
        t = _hswish(dot(wm1_ref[i], h32.astype(bf16)) + bm1_ref[i])
        return t.astype(bf16)

    def s_taps(t):
        taps = []
        mi = 0
        for dy, dx in _TAPS:
            if dy == 0 and dx == 0:
                taps.append(t)
            else:
                s = dy * W + dx
                sh = pltpu.roll(t, (-s) % P, axis=1)
                taps.append(sh * tap_masks[mi])
                mi += 1
        return jnp.concatenate(taps, axis=0)

    def s_m2(i, h32, taps):
        return h32 + _hswish(dot(wm2_ref[i], taps) + bm2_ref[i])

    def s_out(j, h32, z2):
        z1 = _lrelu(dot(wcv3_ref[...], h32.astype(bf16)) + bcspa_ref[...])
        zc = jnp.concatenate([z1, z2], axis=0).astype(bf16)
        o = _hswish(dot(wcv4_ref[...], zc) + bcv4_ref[...])
        out_ref[j] = o.astype(out_ref.dtype)

    hA, zA = s_in(0)
    tA = s_m1(0, hA)
    hB, zB = s_in(1)
    for i in range(n):
        gA = s_taps(tA)
        tB = s_m1(i, hB)
        hA = s_m2(i, hA, gA)
        gB = s_taps(tB)
        if i + 1 < n:
            tA = s_m1(i + 1, hA)
        else:
            s_out(0, hA, zA)
        hB = s_m2(i, hB, gB)
    s_out(1, hB, zB)


def _bn_fold(g, b, m, v, eps=1e-5):
    s = g / jnp.sqrt(v + eps)
    return s, b - m * s


def kernel(x, w_cv1, bn_cv1_g, bn_cv1_b, bn_cv1_m, bn_cv1_v,
           w_m1, bn_m1_g, bn_m1_b, bn_m1_m, bn_m1_v,
           w_m2, bn_m2_g, bn_m2_b, bn_m2_m, bn_m2_v,
           w_cv3, w_cv2, bn_csp_g, bn_csp_b, bn_csp_m, bn_csp_v,
           w_cv4, bn_cv4_g, bn_cv4_b, bn_cv4_m, bn_cv4_v):
    B, c1, H, W = x.shape
    P = H * W
    c_ = w_cv1.shape[0]
    c2 = w_cv4.shape[0]
    n = w_m1.shape[0]
    bf16 = jnp.bfloat16

    s1, b1 = _bn_fold(bn_cv1_g, bn_cv1_b, bn_cv1_m, bn_cv1_v)
    wk_cv1 = w_cv1[:, :, 0, 0] * s1[:, None]
    s_csp, b_csp = _bn_fold(bn_csp_g, bn_csp_b, bn_csp_m, bn_csp_v)
    wk_cv2 = w_cv2[:, :, 0, 0] * s_csp[c_:, None]
    wa = jnp.concatenate([wk_cv1, wk_cv2], axis=0).astype(bf16)
    ba = jnp.concatenate([b1[:, None], b_csp[c_:, None]], axis=0)

    sm1, bm1 = _bn_fold(bn_m1_g, bn_m1_b, bn_m1_m, bn_m1_v)
    wk_m1 = (w_m1[:, :, :, 0, 0] * sm1[:, :, None]).astype(bf16)
    bk_m1 = bm1[:, :, None]

    sm2, bm2 = _bn_fold(bn_m2_g, bn_m2_b, bn_m2_m, bn_m2_v)
    w2f = jnp.transpose(w_m2, (0, 1, 3, 4, 2)).reshape(n, c_, 9 * c_)
    wk_m2 = (w2f * sm2[:, :, None]).astype(bf16)
    bk_m2 = bm2[:, :, None]

    wk_cv3 = (w_cv3[:, :, 0, 0] * s_csp[:c_, None]).astype(bf16)
    bk_cspa = b_csp[:c_, None]

    s4, b4 = _bn_fold(bn_cv4_g, bn_cv4_b, bn_cv4_m, bn_cv4_v)
    wk_cv4 = (w_cv4[:, :, 0, 0] * s4[:, None]).astype(bf16)
    bk_cv4 = b4[:, None]

    pidx = jnp.arange(P, dtype=jnp.int32)
    prow, pcol = pidx // W, pidx % W
    masks = []
    for dy, dx in _TAPS:
        if dy == 0 and dx == 0:
            continue
        ok = ((prow + dy >= 0) & (prow + dy <= H - 1) &
              (pcol + dx >= 0) & (pcol + dx <= W - 1))
        masks.append(ok.astype(bf16).reshape(1, P))
    tap_masks = jnp.stack(masks)

    x_flat = x.reshape(B, c1, P).astype(bf16)

    body = functools.partial(_csp_body, W=W, n=n, c_=c_)
    rep2 = lambda b: (0, 0)
    rep3 = lambda b: (0, 0, 0)

    out_flat = pl.pallas_call(
        body,
        out_shape=jax.ShapeDtypeStruct((B, c2, P), jnp.float32),
        grid_spec=pltpu.PrefetchScalarGridSpec(
            num_scalar_prefetch=0,
            grid=(B // _IMGS,),
            in_specs=[
                pl.BlockSpec((_IMGS, c1, P), lambda b: (b, 0, 0)),
                pl.BlockSpec((8, 1, P), rep3),
                pl.BlockSpec((2 * c_, c1), rep2),
                pl.BlockSpec((2 * c_, 1), rep2),
                pl.BlockSpec((n, c_, c_), rep3),
                pl.BlockSpec((n, c_, 1), rep3),
                pl.BlockSpec((n, c_, 9 * c_), rep3),
                pl.BlockSpec((n, c_, 1), rep3),
                pl.BlockSpec((c_, c_), rep2),
                pl.BlockSpec((c_, 1), rep2),
                pl.BlockSpec((c2, 2 * c_), rep2),
                pl.BlockSpec((c2, 1), rep2),
            ],
            out_specs=pl.BlockSpec((_IMGS, c2, P), lambda b: (b, 0, 0)),
        ),
        compiler_params=pltpu.CompilerParams(
            dimension_semantics=("parallel",),
            vmem_limit_bytes=64 * 1024 * 1024),
    )(x_flat, tap_masks, wa, ba, wk_m1, bk_m1, wk_m2, bk_m2,
      wk_cv3, bk_cspa, wk_cv4, bk_cv4)

    return out_flat.reshape(B, c2, H, W)


# speedup vs baseline: 1.5157x; 1.1122x over previous
import functools

import jax
import jax.numpy as jnp
from jax.experimental import pallas as pl
from jax.experimental.pallas import tpu as pltpu


_TAPS = tuple((dy, dx) for dy in (-1, 0, 1) for dx in (-1, 0, 1))


def _hswish(v):
    return v * jnp.clip(v + 3.0, 0.0, 6.0) * (1.0 / 6.0)


def _hs6(v):
    return v * jnp.clip(v + 3.0, 0.0, 6.0)


def _lrelu(v):
    return jnp.maximum(v, 0.1 * v)


_IMGS = 4


def _csp_body(x_ref, mask_ref, wa_ref, ba_ref, wm1_ref, bm1_ref,
              wm2_ref, bm2_ref, wcv3_ref, bcspa_ref, wcv4_ref, bcv4_ref,
              out_ref, *, W, n, c_):
    f32 = jnp.float32
    bf16 = jnp.bfloat16
    dot = lambda a, b: jnp.dot(a, b, preferred_element_type=f32)
    tap_masks = [mask_ref[k] for k in range(8)]
    P = mask_ref.shape[-1]

    def s_in(j):
        u = (dot(wa_ref[...], x_ref[j]) + ba_ref[...]).astype(bf16)
        return _hs6(u[:c_]), _lrelu(u[c_:])

    def s_m1(i, h):
        return _hs6((dot(wm1_ref[i], h) + bm1_ref[i]).astype(bf16))

    def s_taps(t):
        taps = []
        mi = 0
        for dy, dx in _TAPS:
            if dy == 0 and dx == 0:
                taps.append(t)
            else:
                s = dy * W + dx
                sh = pltpu.roll(t, (-s) % P, axis=1)
                taps.append(sh * tap_masks[mi])
                mi += 1
        return jnp.concatenate(taps, axis=0)

    def s_m2(i, h, taps):
        u = (dot(wm2_ref[i], taps) + bm2_ref[i]).astype(bf16)
        return h + _hs6(u)

    def s_out(j, h, z2):
        z1 = _lrelu((dot(wcv3_ref[...], h) + bcspa_ref[...]).astype(bf16))
        zc = jnp.concatenate([z1, z2], axis=0)
        o = _hswish(dot(wcv4_ref[...], zc) + bcv4_ref[...])
        out_ref[j] = o.astype(out_ref.dtype)

    n_stages = 3 * n + 2
    st = [{} for _ in range(_IMGS)]

    def run_stage(j, s):
        d = st[j]
        if s == 0:
            d['h'], d['z'] = s_in(j)
        elif s == n_stages - 1:
            s_out(j, d['h'], d['z'])
        else:
            i, ph = divmod(s - 1, 3)
            if ph == 0:
                d['t'] = s_m1(i, d['h'])
            elif ph == 1:
                d['g'] = s_taps(d['t'])
            else:
                d['h'] = s_m2(i, d['h'], d['g'])

    for k in range(n_stages + _IMGS - 1):
        for j in range(_IMGS):
            if 0 <= k - j < n_stages:
                run_stage(j, k - j)


def _bn_fold(g, b, m, v, eps=1e-5):
    s = g / jnp.sqrt(v + eps)
    return s, b - m * s


def kernel(x, w_cv1, bn_cv1_g, bn_cv1_b, bn_cv1_m, bn_cv1_v,
           w_m1, bn_m1_g, bn_m1_b, bn_m1_m, bn_m1_v,
           w_m2, bn_m2_g, bn_m2_b, bn_m2_m, bn_m2_v,
           w_cv3, w_cv2, bn_csp_g, bn_csp_b, bn_csp_m, bn_csp_v,
           w_cv4, bn_cv4_g, bn_cv4_b, bn_cv4_m, bn_cv4_v):
    B, c1, H, W = x.shape
    P = H * W
    c_ = w_cv1.shape[0]
    c2 = w_cv4.shape[0]
    n = w_m1.shape[0]
    bf16 = jnp.bfloat16

    s1, b1 = _bn_fold(bn_cv1_g, bn_cv1_b, bn_cv1_m, bn_cv1_v)
    wk_cv1 = w_cv1[:, :, 0, 0] * s1[:, None]
    s_csp, b_csp = _bn_fold(bn_csp_g, bn_csp_b, bn_csp_m, bn_csp_v)
    wk_cv2 = w_cv2[:, :, 0, 0] * s_csp[c_:, None]
    wa = jnp.concatenate([wk_cv1, wk_cv2], axis=0).astype(bf16)
    ba = jnp.concatenate([b1[:, None], b_csp[c_:, None]], axis=0)

    sm1, bm1 = _bn_fold(bn_m1_g, bn_m1_b, bn_m1_m, bn_m1_v)
    wk_m1 = (w_m1[:, :, :, 0, 0] * (sm1 / 6.0)[:, :, None]).astype(bf16)
    bk_m1 = bm1[:, :, None]

    sm2, bm2 = _bn_fold(bn_m2_g, bn_m2_b, bn_m2_m, bn_m2_v)
    w2f = jnp.transpose(w_m2, (0, 1, 3, 4, 2)).reshape(n, c_, 9 * c_)
    wk_m2 = (w2f * (sm2 / 6.0)[:, :, None]).astype(bf16)
    bk_m2 = bm2[:, :, None]

    wk_cv3 = (w_cv3[:, :, 0, 0] * (s_csp[:c_] / 6.0)[:, None]).astype(bf16)
    bk_cspa = b_csp[:c_, None]

    s4, b4 = _bn_fold(bn_cv4_g, bn_cv4_b, bn_cv4_m, bn_cv4_v)
    wk_cv4 = (w_cv4[:, :, 0, 0] * s4[:, None]).astype(bf16)
    bk_cv4 = b4[:, None]

    pidx = jnp.arange(P, dtype=jnp.int32)
    prow, pcol = pidx // W, pidx % W
    masks = []
    for dy, dx in _TAPS:
        if dy == 0 and dx == 0:
            continue
        ok = ((prow + dy >= 0) & (prow + dy <= H - 1) &
              (pcol + dx >= 0) & (pcol + dx <= W - 1))
        masks.append(ok.astype(bf16).reshape(1, P))
    tap_masks = jnp.stack(masks)

    x_flat = x.reshape(B, c1, P).astype(bf16)

    body = functools.partial(_csp_body, W=W, n=n, c_=c_)
    rep2 = lambda b: (0, 0)
    rep3 = lambda b: (0, 0, 0)

    out_flat = pl.pallas_call(
        body,
        out_shape=jax.ShapeDtypeStruct((B, c2, P), jnp.float32),
        grid_spec=pltpu.PrefetchScalarGridSpec(
            num_scalar_prefetch=0,
            grid=(B // _IMGS,),
            in_specs=[
                pl.BlockSpec((_IMGS, c1, P), lambda b: (b, 0, 0)),
                pl.BlockSpec((8, 1, P), rep3),
                pl.BlockSpec((2 * c_, c1), rep2),
                pl.BlockSpec((2 * c_, 1), rep2),
                pl.BlockSpec((n, c_, c_), rep3),
                pl.BlockSpec((n, c_, 1), rep3),
                pl.BlockSpec((n, c_, 9 * c_), rep3),
                pl.BlockSpec((n, c_, 1), rep3),
                pl.BlockSpec((c_, c_), rep2),
                pl.BlockSpec((c_, 1), rep2),
                pl.BlockSpec((c2, 2 * c_), rep2),
                pl.BlockSpec((c2, 1), rep2),
            ],
            out_specs=pl.BlockSpec((_IMGS, c2, P), lambda b: (b, 0, 0)),
        ),
        compiler_params=pltpu.CompilerParams(
            dimension_semantics=("parallel",),
            vmem_limit_bytes=64 * 1024 * 1024),
    )(x_flat, tap_masks, wa, ba, wk_m1, bk_m1, wk_m2, bk_m2,
      wk_cv3, bk_cspa, wk_cv4, bk_cv4)

    return out_flat.reshape(B, c2, H, W)


# speedup vs baseline: 1.5982x; 1.0544x over previous
import functools

import jax
import jax.numpy as jnp
from jax.experimental import pallas as pl
from jax.experimental.pallas import tpu as pltpu


_TAPS = tuple((dy, dx) for dy in (-1, 0, 1) for dx in (-1, 0, 1))


def _hswish(v):
    return v * jnp.clip(v + 3.0, 0.0, 6.0) * (1.0 / 6.0)


def _hs6(v):
    return v * jnp.clip(v + 3.0, 0.0, 6.0)


def _lrelu(v):
    return jnp.maximum(v, 0.1 * v)


_IMGS = 4


def _csp_body(x_ref, mask_ref, wa_ref, ba_ref, wm1_ref, bm1_ref,
              wm2_ref, bm2_ref, wcv3_ref, bcspa_ref, wcv4_ref, bcv4_ref,
              out_ref, *, W, n, c_):
    f32 = jnp.float32
    bf16 = jnp.bfloat16
    dot = lambda a, b: jnp.dot(a, b, preferred_element_type=f32)
    tap_masks = [mask_ref[k] for k in range(8)]
    P = mask_ref.shape[-1]

    def s_in(j):
        u = (dot(wa_ref[...], x_ref[j].astype(bf16)) + ba_ref[...]).astype(bf16)
        return _hs6(u[:c_]), _lrelu(u[c_:])

    def s_m1(i, h):
        return _hs6((dot(wm1_ref[i], h) + bm1_ref[i]).astype(bf16))

    def s_taps(t):
        taps = []
        mi = 0
        for dy, dx in _TAPS:
            if dy == 0 and dx == 0:
                taps.append(t)
            else:
                s = dy * W + dx
                sh = pltpu.roll(t, (-s) % P, axis=1)
                taps.append(sh * tap_masks[mi])
                mi += 1
        return jnp.concatenate(taps, axis=0)

    def s_m2(i, h, taps):
        u = (dot(wm2_ref[i], taps) + bm2_ref[i]).astype(bf16)
        return h + _hs6(u)

    def s_out(j, h, z2):
        z1 = _lrelu((dot(wcv3_ref[...], h) + bcspa_ref[...]).astype(bf16))
        zc = jnp.concatenate([z1, z2], axis=0)
        o = _hswish(dot(wcv4_ref[...], zc) + bcv4_ref[...])
        out_ref[j] = o.astype(out_ref.dtype)

    n_stages = 3 * n + 2
    st = [{} for _ in range(_IMGS)]

    def run_stage(j, s):
        d = st[j]
        if s == 0:
            d['h'], d['z'] = s_in(j)
        elif s == n_stages - 1:
            s_out(j, d['h'], d['z'])
        else:
            i, ph = divmod(s - 1, 3)
            if ph == 0:
                d['t'] = s_m1(i, d['h'])
            elif ph == 1:
                d['g'] = s_taps(d['t'])
            else:
                d['h'] = s_m2(i, d['h'], d['g'])

    for k in range(n_stages + _IMGS - 1):
        for j in range(_IMGS):
            if 0 <= k - j < n_stages:
                run_stage(j, k - j)


def _bn_fold(g, b, m, v, eps=1e-5):
    s = g / jnp.sqrt(v + eps)
    return s, b - m * s


def kernel(x, w_cv1, bn_cv1_g, bn_cv1_b, bn_cv1_m, bn_cv1_v,
           w_m1, bn_m1_g, bn_m1_b, bn_m1_m, bn_m1_v,
           w_m2, bn_m2_g, bn_m2_b, bn_m2_m, bn_m2_v,
           w_cv3, w_cv2, bn_csp_g, bn_csp_b, bn_csp_m, bn_csp_v,
           w_cv4, bn_cv4_g, bn_cv4_b, bn_cv4_m, bn_cv4_v):
    B, c1, H, W = x.shape
    P = H * W
    c_ = w_cv1.shape[0]
    c2 = w_cv4.shape[0]
    n = w_m1.shape[0]
    bf16 = jnp.bfloat16

    s1, b1 = _bn_fold(bn_cv1_g, bn_cv1_b, bn_cv1_m, bn_cv1_v)
    wk_cv1 = w_cv1[:, :, 0, 0] * s1[:, None]
    s_csp, b_csp = _bn_fold(bn_csp_g, bn_csp_b, bn_csp_m, bn_csp_v)
    wk_cv2 = w_cv2[:, :, 0, 0] * s_csp[c_:, None]
    wa = jnp.concatenate([wk_cv1, wk_cv2], axis=0).astype(bf16)
    ba = jnp.concatenate([b1[:, None], b_csp[c_:, None]], axis=0)

    sm1, bm1 = _bn_fold(bn_m1_g, bn_m1_b, bn_m1_m, bn_m1_v)
    wk_m1 = (w_m1[:, :, :, 0, 0] * (sm1 / 6.0)[:, :, None]).astype(bf16)
    bk_m1 = bm1[:, :, None]

    sm2, bm2 = _bn_fold(bn_m2_g, bn_m2_b, bn_m2_m, bn_m2_v)
    w2f = jnp.transpose(w_m2, (0, 1, 3, 4, 2)).reshape(n, c_, 9 * c_)
    wk_m2 = (w2f * (sm2 / 6.0)[:, :, None]).astype(bf16)
    bk_m2 = bm2[:, :, None]

    wk_cv3 = (w_cv3[:, :, 0, 0] * (s_csp[:c_] / 6.0)[:, None]).astype(bf16)
    bk_cspa = b_csp[:c_, None]

    s4, b4 = _bn_fold(bn_cv4_g, bn_cv4_b, bn_cv4_m, bn_cv4_v)
    wk_cv4 = (w_cv4[:, :, 0, 0] * s4[:, None]).astype(bf16)
    bk_cv4 = b4[:, None]

    pidx = jnp.arange(P, dtype=jnp.int32)
    prow, pcol = pidx // W, pidx % W
    masks = []
    for dy, dx in _TAPS:
        if dy == 0 and dx == 0:
            continue
        ok = ((prow + dy >= 0) & (prow + dy <= H - 1) &
              (pcol + dx >= 0) & (pcol + dx <= W - 1))
        masks.append(ok.astype(bf16).reshape(1, P))
    tap_masks = jnp.stack(masks)

    x_flat = x.reshape(B, c1, P)

    body = functools.partial(_csp_body, W=W, n=n, c_=c_)
    rep2 = lambda b: (0, 0)
    rep3 = lambda b: (0, 0, 0)

    out_flat = pl.pallas_call(
        body,
        out_shape=jax.ShapeDtypeStruct((B, c2, P), jnp.float32),
        grid_spec=pltpu.PrefetchScalarGridSpec(
            num_scalar_prefetch=0,
            grid=(B // _IMGS,),
            in_specs=[
                pl.BlockSpec((_IMGS, c1, P), lambda b: (b, 0, 0)),
                pl.BlockSpec((8, 1, P), rep3),
                pl.BlockSpec((2 * c_, c1), rep2),
                pl.BlockSpec((2 * c_, 1), rep2),
                pl.BlockSpec((n, c_, c_), rep3),
                pl.BlockSpec((n, c_, 1), rep3),
                pl.BlockSpec((n, c_, 9 * c_), rep3),
                pl.BlockSpec((n, c_, 1), rep3),
                pl.BlockSpec((c_, c_), rep2),
                pl.BlockSpec((c_, 1), rep2),
                pl.BlockSpec((c2, 2 * c_), rep2),
                pl.BlockSpec((c2, 1), rep2),
            ],
            out_specs=pl.BlockSpec((_IMGS, c2, P), lambda b: (b, 0, 0)),
        ),
        compiler_params=pltpu.CompilerParams(
            dimension_semantics=("parallel",),
            vmem_limit_bytes=64 * 1024 * 1024),
    )(x_flat, tap_masks, wa, ba, wk_m1, bk_m1, wk_m2, bk_m2,
      wk_cv3, bk_cspa, wk_cv4, bk_cv4)

    return out_flat.reshape(B, c2, H, W)


# speedup vs baseline: 1.6857x; 1.0547x over previous
import functools

import jax
import jax.numpy as jnp
from jax.experimental import pallas as pl
from jax.experimental.pallas import tpu as pltpu


_TAPS = tuple((dy, dx) for dy in (-1, 0, 1) for dx in (-1, 0, 1))


def _hswish(v):
    return v * jnp.clip(v + 3.0, 0.0, 6.0) * (1.0 / 6.0)


def _hs6(v):
    return v * jnp.clip(v + 3.0, 0.0, 6.0)


def _lrelu(v):
    return jnp.maximum(v, 0.1 * v)


_IMGS = 8


def _csp_body(x_ref, mask_ref, wa_ref, ba_ref, wm1_ref, bm1_ref,
              wm2_ref, wcv3_ref, bcspa_ref, wcv4_ref, bcv4_ref,
              out_ref, *, W, n, c_):
    f32 = jnp.float32
    bf16 = jnp.bfloat16
    dot = lambda a, b: jnp.dot(a, b, preferred_element_type=f32)
    tap_masks = [mask_ref[k] for k in range(8)]
    P = mask_ref.shape[-1]

    def s_in(j):
        u = (dot(wa_ref[...], x_ref[j].astype(bf16)) + ba_ref[...]).astype(bf16)
        return _hs6(u[:c_]), _lrelu(u[c_:])

    def s_m1(i, h):
        return _hs6((dot(wm1_ref[i], h) + bm1_ref[i]).astype(bf16))

    row_idx = jax.lax.broadcasted_iota(jnp.int32, (16, P), 0)
    ones_blk = jnp.maximum(1 - row_idx, 0).astype(bf16)

    def s_taps(j, t):
        taps = []
        mi = 0
        for dy, dx in _TAPS:
            if dy == 0 and dx == 0:
                taps.append(t)
            else:
                s = dy * W + dx
                sh = pltpu.roll(t, (-s) % P, axis=1)
                taps.append(sh * tap_masks[mi])
                mi += 1
        taps.append(ones_blk)
        return jnp.concatenate(taps, axis=0)

    def s_m2(i, h, taps):
        u = dot(wm2_ref[i], taps).astype(bf16)
        return h + _hs6(u)

    def s_out(j, h, z2):
        z1 = _lrelu((dot(wcv3_ref[...], h) + bcspa_ref[...]).astype(bf16))
        zc = jnp.concatenate([z1, z2], axis=0)
        u6 = (dot(wcv4_ref[...], zc) + bcv4_ref[...]).astype(bf16)
        o = u6 * jnp.clip(6.0 * u6 + 3.0, 0.0, 6.0)
        out_ref[j] = o.astype(out_ref.dtype)

    n_stages = 3 * n + 2
    st = [{} for _ in range(_IMGS)]

    def run_stage(j, s):
        d = st[j]
        if s == 0:
            d['h'], d['z'] = s_in(j)
        elif s == n_stages - 1:
            s_out(j, d['h'], d['z'])
        else:
            i, ph = divmod(s - 1, 3)
            if ph == 0:
                d['t'] = s_m1(i, d['h'])
            elif ph == 1:
                d['g'] = s_taps(j, d['t'])
            else:
                d['h'] = s_m2(i, d['h'], d['g'])

    for k in range(n_stages + _IMGS - 1):
        for j in range(_IMGS):
            if 0 <= k - j < n_stages:
                run_stage(j, k - j)


def _bn_fold(g, b, m, v, eps=1e-5):
    s = g / jnp.sqrt(v + eps)
    return s, b - m * s


def kernel(x, w_cv1, bn_cv1_g, bn_cv1_b, bn_cv1_m, bn_cv1_v,
           w_m1, bn_m1_g, bn_m1_b, bn_m1_m, bn_m1_v,
           w_m2, bn_m2_g, bn_m2_b, bn_m2_m, bn_m2_v,
           w_cv3, w_cv2, bn_csp_g, bn_csp_b, bn_csp_m, bn_csp_v,
           w_cv4, bn_cv4_g, bn_cv4_b, bn_cv4_m, bn_cv4_v):
    B, c1, H, W = x.shape
    P = H * W
    c_ = w_cv1.shape[0]
    c2 = w_cv4.shape[0]
    n = w_m1.shape[0]
    bf16 = jnp.bfloat16

    s1, b1 = _bn_fold(bn_cv1_g, bn_cv1_b, bn_cv1_m, bn_cv1_v)
    wk_cv1 = w_cv1[:, :, 0, 0] * s1[:, None]
    s_csp, b_csp = _bn_fold(bn_csp_g, bn_csp_b, bn_csp_m, bn_csp_v)
    wk_cv2 = w_cv2[:, :, 0, 0] * s_csp[c_:, None]
    wa = jnp.concatenate([wk_cv1, wk_cv2], axis=0).astype(bf16)
    ba = jnp.concatenate([b1[:, None], b_csp[c_:, None]], axis=0)

    sm1, bm1 = _bn_fold(bn_m1_g, bn_m1_b, bn_m1_m, bn_m1_v)
    wk_m1 = (w_m1[:, :, :, 0, 0] * (sm1 / 6.0)[:, :, None]).astype(bf16)
    bk_m1 = bm1[:, :, None]

    sm2, bm2 = _bn_fold(bn_m2_g, bn_m2_b, bn_m2_m, bn_m2_v)
    w2f = jnp.transpose(w_m2, (0, 1, 3, 4, 2)).reshape(n, c_, 9 * c_)
    w2aug = jnp.concatenate(
        [w2f * (sm2 / 6.0)[:, :, None], bm2[:, :, None],
         jnp.zeros((n, c_, 15), jnp.float32)], axis=2)
    wk_m2 = w2aug.astype(bf16)

    wk_cv3 = (w_cv3[:, :, 0, 0] * (s_csp[:c_] / 6.0)[:, None]).astype(bf16)
    bk_cspa = b_csp[:c_, None]

    s4, b4 = _bn_fold(bn_cv4_g, bn_cv4_b, bn_cv4_m, bn_cv4_v)
    wk_cv4 = (w_cv4[:, :, 0, 0] * (s4 / 6.0)[:, None]).astype(bf16)
    bk_cv4 = (b4 / 6.0)[:, None]

    pidx = jnp.arange(P, dtype=jnp.int32)
    prow, pcol = pidx // W, pidx % W
    masks = []
    for dy, dx in _TAPS:
        if dy == 0 and dx == 0:
            continue
        ok = ((prow + dy >= 0) & (prow + dy <= H - 1) &
              (pcol + dx >= 0) & (pcol + dx <= W - 1))
        masks.append(ok.astype(bf16).reshape(1, P))
    tap_masks = jnp.stack(masks)

    x_flat = x.reshape(B, c1, P)

    body = functools.partial(_csp_body, W=W, n=n, c_=c_)
    rep2 = lambda b: (0, 0)
    rep3 = lambda b: (0, 0, 0)

    out_flat = pl.pallas_call(
        body,
        out_shape=jax.ShapeDtypeStruct((B, c2, P), jnp.float32),
        grid_spec=pltpu.PrefetchScalarGridSpec(
            num_scalar_prefetch=0,
            grid=(B // _IMGS,),
            in_specs=[
                pl.BlockSpec((_IMGS, c1, P), lambda b: (b, 0, 0)),
                pl.BlockSpec((8, 1, P), rep3),
                pl.BlockSpec((2 * c_, c1), rep2),
                pl.BlockSpec((2 * c_, 1), rep2),
                pl.BlockSpec((n, c_, c_), rep3),
                pl.BlockSpec((n, c_, 1), rep3),
                pl.BlockSpec((n, c_, 9 * c_ + 16), rep3),
                pl.BlockSpec((c_, c_), rep2),
                pl.BlockSpec((c_, 1), rep2),
                pl.BlockSpec((c2, 2 * c_), rep2),
                pl.BlockSpec((c2, 1), rep2),
            ],
            out_specs=pl.BlockSpec((_IMGS, c2, P), lambda b: (b, 0, 0)),
        ),
        compiler_params=pltpu.CompilerParams(
            dimension_semantics=("parallel",),
            vmem_limit_bytes=64 * 1024 * 1024),
    )(x_flat, tap_masks, wa, ba, wk_m1, bk_m1, wk_m2,
      wk_cv3, bk_cspa, wk_cv4, bk_cv4)

    return out_flat.reshape(B, c2, H, W)


# speedup vs baseline: 1.6869x; 1.0007x over previous
import functools

import jax
import jax.numpy as jnp
from jax.experimental import pallas as pl
from jax.experimental.pallas import tpu as pltpu


_TAPS = tuple((dy, dx) for dy in (-1, 0, 1) for dx in (-1, 0, 1))


def _hswish(v):
    return v * jnp.clip(v + 3.0, 0.0, 6.0) * (1.0 / 6.0)


def _hs6(v):
    return v * jnp.clip(v + 3.0, 0.0, 6.0)


def _lrelu(v):
    return jnp.maximum(v, 0.1 * v)


_IMGS = 8


def _csp_body(x_ref, mask_ref, wa_ref, ba_ref, wm1_ref, bm1_ref,
              wm2_ref, wcv3_ref, bcspa_ref, wcv4_ref, bcv4_ref,
              out_ref, *, W, n, c_, imgs):
    f32 = jnp.float32
    bf16 = jnp.bfloat16
    dot = lambda a, b: jnp.dot(a, b, preferred_element_type=f32)
    tap_masks = [mask_ref[k] for k in range(8)]
    P = mask_ref.shape[-1]

    def s_in(j):
        u = (dot(wa_ref[...], x_ref[j].astype(bf16)) + ba_ref[...]).astype(bf16)
        return _hs6(u[:c_]), _lrelu(u[c_:])

    def s_m1(i, h):
        return _hs6((dot(wm1_ref[i], h) + bm1_ref[i]).astype(bf16))

    row_idx = jax.lax.broadcasted_iota(jnp.int32, (16, P), 0)
    ones_blk = jnp.maximum(1 - row_idx, 0).astype(bf16)

    def s_taps(j, t):
        taps = []
        mi = 0
        for dy, dx in _TAPS:
            if dy == 0 and dx == 0:
                taps.append(t)
            else:
                s = dy * W + dx
                sh = pltpu.roll(t, (-s) % P, axis=1)
                taps.append(sh * tap_masks[mi])
                mi += 1
        taps.append(ones_blk)
        return jnp.concatenate(taps, axis=0)

    def s_m2(i, h, taps):
        u = dot(wm2_ref[i], taps).astype(bf16)
        return h + _hs6(u)

    def s_out(j, h, z2):
        z1 = _lrelu((dot(wcv3_ref[...], h) + bcspa_ref[...]).astype(bf16))
        zc = jnp.concatenate([z1, z2], axis=0)
        u6 = (dot(wcv4_ref[...], zc) + bcv4_ref[...]).astype(bf16)
        o = u6 * jnp.clip(6.0 * u6 + 3.0, 0.0, 6.0)
        out_ref[j] = o.astype(out_ref.dtype)

    n_stages = 3 * n + 2
    st = [{} for _ in range(imgs)]

    def run_stage(j, s):
        d = st[j]
        if s == 0:
            d['h'], d['z'] = s_in(j)
        elif s == n_stages - 1:
            s_out(j, d['h'], d['z'])
        else:
            i, ph = divmod(s - 1, 3)
            if ph == 0:
                d['t'] = s_m1(i, d['h'])
            elif ph == 1:
                d['g'] = s_taps(j, d['t'])
            else:
                d['h'] = s_m2(i, d['h'], d['g'])

    for k in range(n_stages + imgs - 1):
        for j in range(imgs):
            if 0 <= k - j < n_stages:
                run_stage(j, k - j)


def _bn_fold(g, b, m, v, eps=1e-5):
    s = g / jnp.sqrt(v + eps)
    return s, b - m * s


def kernel(x, w_cv1, bn_cv1_g, bn_cv1_b, bn_cv1_m, bn_cv1_v,
           w_m1, bn_m1_g, bn_m1_b, bn_m1_m, bn_m1_v,
           w_m2, bn_m2_g, bn_m2_b, bn_m2_m, bn_m2_v,
           w_cv3, w_cv2, bn_csp_g, bn_csp_b, bn_csp_m, bn_csp_v,
           w_cv4, bn_cv4_g, bn_cv4_b, bn_cv4_m, bn_cv4_v):
    B, c1, H, W = x.shape
    P = H * W
    c_ = w_cv1.shape[0]
    c2 = w_cv4.shape[0]
    n = w_m1.shape[0]
    bf16 = jnp.bfloat16

    s1, b1 = _bn_fold(bn_cv1_g, bn_cv1_b, bn_cv1_m, bn_cv1_v)
    wk_cv1 = w_cv1[:, :, 0, 0] * s1[:, None]
    s_csp, b_csp = _bn_fold(bn_csp_g, bn_csp_b, bn_csp_m, bn_csp_v)
    wk_cv2 = w_cv2[:, :, 0, 0] * s_csp[c_:, None]
    wa = jnp.concatenate([wk_cv1, wk_cv2], axis=0).astype(bf16)
    ba = jnp.concatenate([b1[:, None], b_csp[c_:, None]], axis=0)

    sm1, bm1 = _bn_fold(bn_m1_g, bn_m1_b, bn_m1_m, bn_m1_v)
    wk_m1 = (w_m1[:, :, :, 0, 0] * (sm1 / 6.0)[:, :, None]).astype(bf16)
    bk_m1 = bm1[:, :, None]

    sm2, bm2 = _bn_fold(bn_m2_g, bn_m2_b, bn_m2_m, bn_m2_v)
    w2f = jnp.transpose(w_m2, (0, 1, 3, 4, 2)).reshape(n, c_, 9 * c_)
    w2aug = jnp.concatenate(
        [w2f * (sm2 / 6.0)[:, :, None], bm2[:, :, None],
         jnp.zeros((n, c_, 15), jnp.float32)], axis=2)
    wk_m2 = w2aug.astype(bf16)

    wk_cv3 = (w_cv3[:, :, 0, 0] * (s_csp[:c_] / 6.0)[:, None]).astype(bf16)
    bk_cspa = b_csp[:c_, None]

    s4, b4 = _bn_fold(bn_cv4_g, bn_cv4_b, bn_cv4_m, bn_cv4_v)
    wk_cv4 = (w_cv4[:, :, 0, 0] * (s4 / 6.0)[:, None]).astype(bf16)
    bk_cv4 = (b4 / 6.0)[:, None]

    pidx = jnp.arange(P, dtype=jnp.int32)
    prow, pcol = pidx // W, pidx % W
    masks = []
    for dy, dx in _TAPS:
        if dy == 0 and dx == 0:
            continue
        ok = ((prow + dy >= 0) & (prow + dy <= H - 1) &
              (pcol + dx >= 0) & (pcol + dx <= W - 1))
        masks.append(ok.astype(bf16).reshape(1, P))
    tap_masks = jnp.stack(masks)

    x_flat = x.reshape(B, c1, P)

    imgs = next(g for g in (_IMGS, 4, 2, 1) if g <= _IMGS and B % g == 0)
    body = functools.partial(_csp_body, W=W, n=n, c_=c_, imgs=imgs)
    rep2 = lambda b: (0, 0)
    rep3 = lambda b: (0, 0, 0)

    out_flat = pl.pallas_call(
        body,
        out_shape=jax.ShapeDtypeStruct((B, c2, P), jnp.float32),
        grid_spec=pltpu.PrefetchScalarGridSpec(
            num_scalar_prefetch=0,
            grid=(B // imgs,),
            in_specs=[
                pl.BlockSpec((imgs, c1, P), lambda b: (b, 0, 0)),
                pl.BlockSpec((8, 1, P), rep3),
                pl.BlockSpec((2 * c_, c1), rep2),
                pl.BlockSpec((2 * c_, 1), rep2),
                pl.BlockSpec((n, c_, c_), rep3),
                pl.BlockSpec((n, c_, 1), rep3),
                pl.BlockSpec((n, c_, 9 * c_ + 16), rep3),
                pl.BlockSpec((c_, c_), rep2),
                pl.BlockSpec((c_, 1), rep2),
                pl.BlockSpec((c2, 2 * c_), rep2),
                pl.BlockSpec((c2, 1), rep2),
            ],
            out_specs=pl.BlockSpec((imgs, c2, P), lambda b: (b, 0, 0)),
        ),
        compiler_params=pltpu.CompilerParams(
            dimension_semantics=("parallel",),
            vmem_limit_bytes=64 * 1024 * 1024),
    )(x_flat, tap_masks, wa, ba, wk_m1, bk_m1, wk_m2,
      wk_cv3, bk_cspa, wk_cv4, bk_cv4)

    return out_flat.reshape(B, c2, H, W)
